# baseline (speedup 1.0000x reference)
import functools

import jax
import jax.numpy as jnp
from jax import lax
from jax.experimental import pallas as pl
from jax.experimental.pallas import tpu as pltpu
from jax.experimental.pallas import tpu_sc as plsc

N = 10000
E = 160000
D = 128

NC = 2
NS = 16
NW = NC * NS
EPW = E // NW
CH = 125
NCHUNK = EPW // CH


_NP = 10240
_DCW = 640


def _deg_body(dst_hbm, out_hbm, deg_v, idx_v, red_v, acc_v, slab_sh):
    c = lax.axis_index("c")
    s = lax.axis_index("s")
    wid = s * NC + c

    zeros16 = jnp.zeros((16,), jnp.float32)

    def zero_body(i, _):
        deg_v[pl.ds(i * 16, 16)] = zeros16
        return 0

    lax.fori_loop(0, _NP // 16, zero_body, 0)

    pltpu.sync_copy(dst_hbm.at[wid], idx_v)

    ones16 = jnp.ones((16,), jnp.float32)
    nfull = EPW // 16

    def scat_body(i, _):
        idx = idx_v[pl.ds(i * 16, 16)]
        plsc.addupdate_scatter(deg_v, [idx], ones16)
        return 0

    lax.fori_loop(0, nfull, scat_body, 0)

    rem = EPW - nfull * 16
    if rem:
        idx = idx_v[pl.ds(EPW - 16, 16)]
        mask = lax.iota(jnp.int32, 16) >= (16 - rem)
        plsc.addupdate_scatter(deg_v, [idx], ones16, mask=mask)

    pltpu.sync_copy(deg_v, slab_sh.at[s])
    plsc.subcore_barrier()

    base = s * _DCW
    pltpu.sync_copy(slab_sh.at[:, pl.ds(base, _DCW)], red_v)
    for k in range(_DCW // 16):
        acc = red_v[0, pl.ds(k * 16, 16)]
        for r in range(1, NS):
            acc = acc + red_v[r, pl.ds(k * 16, 16)]
        acc_v[pl.ds(k * 16, 16)] = acc
    pltpu.sync_copy(acc_v, out_hbm.at[c, pl.ds(base, _DCW)])


def _msg_body(u_hbm, src_hbm, dst_hbm, out_hbm, src_v, dst_v, rows0_v,
              acc_sh, sem0):
    c = lax.axis_index("c")
    s = lax.axis_index("s")
    wid = s * NC + c

    zeros16 = jnp.zeros((16,), jnp.float32)

    def zrow(r, _):
        for j in range(D // 16):
            rows0_v[r, pl.ds(j * 16, 16)] = zeros16
        return 0

    lax.fori_loop(0, CH, zrow, 0)

    @pl.when(s < 10)
    def _():
        for k in range(8):
            pltpu.sync_copy(rows0_v, acc_sh.at[pl.ds(s * 1000 + k * CH, CH)])

    pltpu.sync_copy(src_hbm.at[wid], src_v)
    pltpu.sync_copy(dst_hbm.at[wid], dst_v)

    plsc.subcore_barrier()

    def chunk_body(j, _):
        pltpu.async_copy(u_hbm.at[src_v.at[j]], rows0_v, sem0).wait()
        pltpu.sync_copy(rows0_v, acc_sh.at[dst_v.at[j]], add=True)
        return 0

    lax.fori_loop(0, NCHUNK, chunk_body, 0)

    plsc.subcore_barrier()

    @pl.when(s < 10)
    def _():
        pltpu.sync_copy(
            acc_sh.at[pl.ds(s * 1000, 1000)],
            out_hbm.at[c, pl.ds(s * 1000, 1000)],
        )


@functools.cache
def _sc_kernels():
    mesh = plsc.VectorSubcoreMesh(
        core_axis_name="c", subcore_axis_name="s", num_cores=NC, num_subcores=NS
    )
    params = pltpu.CompilerParams(needs_layout_passes=False)
    deg_k = pl.kernel(
        _deg_body,
        out_type=jax.ShapeDtypeStruct((NC, _NP), jnp.float32),
        mesh=mesh,
        compiler_params=params,
        scratch_types=[
            pltpu.VMEM((_NP,), jnp.float32),
            pltpu.VMEM((EPW,), jnp.int32),
            pltpu.VMEM((NS, _DCW), jnp.float32),
            pltpu.VMEM((_DCW,), jnp.float32),
            pltpu.VMEM_SHARED((NS, _NP), jnp.float32),
        ],
    )
    msg_k = pl.kernel(
        _msg_body,
        out_type=jax.ShapeDtypeStruct((NC, N, D), jnp.float32),
        mesh=mesh,
        compiler_params=params,
        scratch_types=[
            pltpu.VMEM((NCHUNK, CH), jnp.int32),
            pltpu.VMEM((NCHUNK, CH), jnp.int32),
            pltpu.VMEM((CH, D), jnp.float32),
            pltpu.VMEM_SHARED((N, D), jnp.float32),
            pltpu.SemaphoreType.DMA,
        ],
    )
    return deg_k, msg_k


_RB = 1000


def _h_body(x_ref, w_ref, h_ref):
    h_ref[...] = jnp.dot(x_ref[...], w_ref[...],
                         preferred_element_type=jnp.float32)


def _h_call(x, W):
    return pl.pallas_call(
        _h_body,
        grid=(N // _RB,),
        in_specs=[
            pl.BlockSpec((_RB, D), lambda i: (i, 0)),
            pl.BlockSpec((D, D), lambda i: (0, 0)),
        ],
        out_specs=pl.BlockSpec((_RB, D), lambda i: (i, 0)),
        out_shape=jax.ShapeDtypeStruct((N, D), jnp.float32),
    )(x, W)


def _u_body(h_ref, d0_ref, d1_ref, u_ref, dinv_ref):
    dinv = lax.rsqrt(d0_ref[...] + d1_ref[...] + 1.0)
    dinv_ref[...] = dinv
    u_ref[...] = h_ref[...] * dinv


def _u_call(h, d0, d1):
    return pl.pallas_call(
        _u_body,
        grid=(N // _RB,),
        in_specs=[
            pl.BlockSpec((_RB, D), lambda i: (i, 0)),
            pl.BlockSpec((_RB, 1), lambda i: (i, 0)),
            pl.BlockSpec((_RB, 1), lambda i: (i, 0)),
        ],
        out_specs=[
            pl.BlockSpec((_RB, D), lambda i: (i, 0)),
            pl.BlockSpec((_RB, 1), lambda i: (i, 0)),
        ],
        out_shape=[
            jax.ShapeDtypeStruct((N, D), jnp.float32),
            jax.ShapeDtypeStruct((N, 1), jnp.float32),
        ],
    )(h, d0, d1)


_RSB = 400


def _rowsum_body(a_ref, degA_ref, tot_ref):
    i = pl.program_id(0)
    part = jnp.sum(a_ref[...], axis=1, keepdims=True)
    degA_ref[...] = part
    blk = jnp.sum(part).reshape(1, 1)

    @pl.when(i == 0)
    def _():
        tot_ref[...] = blk

    @pl.when(i > 0)
    def _():
        tot_ref[...] += blk


def _rowsum_call(A):
    return pl.pallas_call(
        _rowsum_body,
        grid=(N // _RSB,),
        in_specs=[pl.BlockSpec((_RSB, N), lambda i: (i, 0))],
        out_specs=[
            pl.BlockSpec((_RSB, 1), lambda i: (i, 0)),
            pl.BlockSpec((1, 1), lambda i: (0, 0)),
        ],
        out_shape=[
            jax.ShapeDtypeStruct((N, 1), jnp.float32),
            jax.ShapeDtypeStruct((1, 1), jnp.float32),
        ],
        compiler_params=pltpu.CompilerParams(
            dimension_semantics=("arbitrary",),
        ),
    )(A)


def _final_body(u_ref, acc_ref, dinv_ref, degA_ref, tot_ref, w_ref, t_ref,
                b_ref, out_ref):
    gcn = dinv_ref[...] * (acc_ref[0] + acc_ref[1] + u_ref[...]) + b_ref[...]

    T = t_ref[...]
    z = gcn / T
    m = jnp.max(z, axis=1, keepdims=True)
    e = jnp.exp(z - m)
    ssum = jnp.sum(e, axis=1, keepdims=True)
    p = e / ssum
    logp = jnp.log(p + 1e-12)
    H = -jnp.sum(p * logp, axis=1, keepdims=True)
    g = -(p * (logp + H)) / T
    wn = degA_ref[...] / (tot_ref[...] + 1e-12)
    out_ref[...] = gcn + w_ref[...] * (g * wn)


def _final_call(u, accp, dinv, degA, tot, weight, temperature, b):
    return pl.pallas_call(
        _final_body,
        grid=(N // _RB,),
        in_specs=[
            pl.BlockSpec((_RB, D), lambda i: (i, 0)),
            pl.BlockSpec((NC, _RB, D), lambda i: (0, i, 0)),
            pl.BlockSpec((_RB, 1), lambda i: (i, 0)),
            pl.BlockSpec((_RB, 1), lambda i: (i, 0)),
            pl.BlockSpec((1, 1), lambda i: (0, 0)),
            pl.BlockSpec((1, 1), lambda i: (0, 0)),
            pl.BlockSpec((1, 1), lambda i: (0, 0)),
            pl.BlockSpec((1, D), lambda i: (0, 0)),
        ],
        out_specs=pl.BlockSpec((_RB, D), lambda i: (i, 0)),
        out_shape=jax.ShapeDtypeStruct((N, D), jnp.float32),
    )(u, accp, dinv, degA, tot, weight, temperature, b)


@jax.jit
def kernel(x, edge_index, A, weight, temperature, W, b):
    src = edge_index[0].reshape(NW, NCHUNK, CH)
    dst = edge_index[1]

    deg_k, msg_k = _sc_kernels()
    h = _h_call(x, W)
    degp = deg_k(dst.reshape(NW, EPW))
    u, dinv = _u_call(h, degp[0, :N].reshape(N, 1), degp[1, :N].reshape(N, 1))
    accp = msg_k(u, src, dst.reshape(NW, NCHUNK, CH))
    degA, tot = _rowsum_call(A)
    return _final_call(
        u, accp, dinv, degA, tot,
        weight.reshape(1, 1), temperature.reshape(1, 1), b.reshape(1, D),
    )


# speedup vs baseline: 1.0419x; 1.0419x over previous
import functools

import jax
import jax.numpy as jnp
from jax import lax
from jax.experimental import pallas as pl
from jax.experimental.pallas import tpu as pltpu
from jax.experimental.pallas import tpu_sc as plsc

N = 10000
E = 160000
D = 128

NC = 2
NS = 16
NW = NC * NS
EPW = E // NW
CH = 125
NCHUNK = EPW // CH


def _deg_body(dst_hbm, out_hbm, deg_v, idx_v):
    c = lax.axis_index("c")
    s = lax.axis_index("s")
    wid = s * NC + c

    zeros16 = jnp.zeros((16,), jnp.float32)

    def zero_body(i, _):
        deg_v[pl.ds(i * 16, 16)] = zeros16
        return 0

    lax.fori_loop(0, N // 16, zero_body, 0)

    pltpu.sync_copy(dst_hbm.at[wid], idx_v)

    ones16 = jnp.ones((16,), jnp.float32)
    nfull = EPW // 16

    def scat_body(i, _):
        idx = idx_v[pl.ds(i * 16, 16)]
        plsc.addupdate_scatter(deg_v, [idx], ones16)
        return 0

    lax.fori_loop(0, nfull, scat_body, 0)

    rem = EPW - nfull * 16
    if rem:
        idx = idx_v[pl.ds(EPW - 16, 16)]
        mask = lax.iota(jnp.int32, 16) >= (16 - rem)
        plsc.addupdate_scatter(deg_v, [idx], ones16, mask=mask)

    pltpu.sync_copy(deg_v, out_hbm.at[wid])


def _msg_body(u_hbm, src_hbm, dst_hbm, out_hbm, src_v, dst_v, rows0_v,
              acc_sh, sem0):
    c = lax.axis_index("c")
    s = lax.axis_index("s")
    wid = s * NC + c

    zeros16 = jnp.zeros((16,), jnp.float32)

    def zrow(r, _):
        for j in range(D // 16):
            rows0_v[r, pl.ds(j * 16, 16)] = zeros16
        return 0

    lax.fori_loop(0, CH, zrow, 0)

    @pl.when(s < 10)
    def _():
        for k in range(8):
            pltpu.sync_copy(rows0_v, acc_sh.at[pl.ds(s * 1000 + k * CH, CH)])

    pltpu.sync_copy(src_hbm.at[wid], src_v)
    pltpu.sync_copy(dst_hbm.at[wid], dst_v)

    plsc.subcore_barrier()

    def chunk_body(j, _):
        pltpu.async_copy(u_hbm.at[src_v.at[j]], rows0_v, sem0).wait()
        pltpu.sync_copy(rows0_v, acc_sh.at[dst_v.at[j]], add=True)
        return 0

    lax.fori_loop(0, NCHUNK, chunk_body, 0)

    plsc.subcore_barrier()

    @pl.when(s < 10)
    def _():
        pltpu.sync_copy(
            acc_sh.at[pl.ds(s * 1000, 1000)],
            out_hbm.at[c, pl.ds(s * 1000, 1000)],
        )


@functools.cache
def _sc_kernels():
    mesh = plsc.VectorSubcoreMesh(
        core_axis_name="c", subcore_axis_name="s", num_cores=NC, num_subcores=NS
    )
    params = pltpu.CompilerParams(needs_layout_passes=False)
    deg_k = pl.kernel(
        _deg_body,
        out_type=jax.ShapeDtypeStruct((NW, N), jnp.float32),
        mesh=mesh,
        compiler_params=params,
        scratch_types=[
            pltpu.VMEM((N,), jnp.float32),
            pltpu.VMEM((EPW,), jnp.int32),
        ],
    )
    msg_k = pl.kernel(
        _msg_body,
        out_type=jax.ShapeDtypeStruct((NC, N, D), jnp.float32),
        mesh=mesh,
        compiler_params=params,
        scratch_types=[
            pltpu.VMEM((NCHUNK, CH), jnp.int32),
            pltpu.VMEM((NCHUNK, CH), jnp.int32),
            pltpu.VMEM((CH, D), jnp.float32),
            pltpu.VMEM_SHARED((N, D), jnp.float32),
            pltpu.SemaphoreType.DMA,
        ],
    )
    return deg_k, msg_k


_RB = 1000


def _dinv_body(degp_ref, dinv_ref):
    deg = jnp.sum(degp_ref[...], axis=0) + 1.0
    dinv_ref[...] = lax.rsqrt(deg)[:, None]


def _dinv_call(degp):
    return pl.pallas_call(
        _dinv_body,
        out_shape=jax.ShapeDtypeStruct((N, 1), jnp.float32),
    )(degp)


def _u_body(x_ref, w_ref, dinv_ref, u_ref):
    h = jnp.dot(x_ref[...], w_ref[...], preferred_element_type=jnp.float32)
    u_ref[...] = h * dinv_ref[...]


def _u_call(x, W, dinv):
    return pl.pallas_call(
        _u_body,
        grid=(N // _RB,),
        in_specs=[
            pl.BlockSpec((_RB, D), lambda i: (i, 0)),
            pl.BlockSpec((D, D), lambda i: (0, 0)),
            pl.BlockSpec((_RB, 1), lambda i: (i, 0)),
        ],
        out_specs=pl.BlockSpec((_RB, D), lambda i: (i, 0)),
        out_shape=jax.ShapeDtypeStruct((N, D), jnp.float32),
    )(x, W, dinv)


_RSB = 400


def _rowsum_body(a_ref, degA_ref, tot_ref):
    i = pl.program_id(0)
    part = jnp.sum(a_ref[...], axis=1, keepdims=True)
    degA_ref[...] = part
    blk = jnp.sum(part).reshape(1, 1)

    @pl.when(i == 0)
    def _():
        tot_ref[...] = blk

    @pl.when(i > 0)
    def _():
        tot_ref[...] += blk


def _rowsum_call(A):
    return pl.pallas_call(
        _rowsum_body,
        grid=(N // _RSB,),
        in_specs=[pl.BlockSpec((_RSB, N), lambda i: (i, 0))],
        out_specs=[
            pl.BlockSpec((_RSB, 1), lambda i: (i, 0)),
            pl.BlockSpec((1, 1), lambda i: (0, 0)),
        ],
        out_shape=[
            jax.ShapeDtypeStruct((N, 1), jnp.float32),
            jax.ShapeDtypeStruct((1, 1), jnp.float32),
        ],
        compiler_params=pltpu.CompilerParams(
            dimension_semantics=("arbitrary",),
        ),
    )(A)


def _final_body(u_ref, acc_ref, dinv_ref, degA_ref, tot_ref, w_ref, t_ref,
                b_ref, out_ref):
    gcn = dinv_ref[...] * (acc_ref[0] + acc_ref[1] + u_ref[...]) + b_ref[...]

    T = t_ref[...]
    z = gcn / T
    m = jnp.max(z, axis=1, keepdims=True)
    e = jnp.exp(z - m)
    ssum = jnp.sum(e, axis=1, keepdims=True)
    p = e / ssum
    logp = jnp.log(p + 1e-12)
    H = -jnp.sum(p * logp, axis=1, keepdims=True)
    g = -(p * (logp + H)) / T
    wn = degA_ref[...] / (tot_ref[...] + 1e-12)
    out_ref[...] = gcn + w_ref[...] * (g * wn)


def _final_call(u, accp, dinv, degA, tot, weight, temperature, b):
    return pl.pallas_call(
        _final_body,
        grid=(N // _RB,),
        in_specs=[
            pl.BlockSpec((_RB, D), lambda i: (i, 0)),
            pl.BlockSpec((NC, _RB, D), lambda i: (0, i, 0)),
            pl.BlockSpec((_RB, 1), lambda i: (i, 0)),
            pl.BlockSpec((_RB, 1), lambda i: (i, 0)),
            pl.BlockSpec((1, 1), lambda i: (0, 0)),
            pl.BlockSpec((1, 1), lambda i: (0, 0)),
            pl.BlockSpec((1, 1), lambda i: (0, 0)),
            pl.BlockSpec((1, D), lambda i: (0, 0)),
        ],
        out_specs=pl.BlockSpec((_RB, D), lambda i: (i, 0)),
        out_shape=jax.ShapeDtypeStruct((N, D), jnp.float32),
    )(u, accp, dinv, degA, tot, weight, temperature, b)


@jax.jit
def kernel(x, edge_index, A, weight, temperature, W, b):
    src = edge_index[0].reshape(NW, NCHUNK, CH)
    dst = edge_index[1]

    deg_k, msg_k = _sc_kernels()
    degp = deg_k(dst.reshape(NW, EPW))
    dinv = _dinv_call(degp)
    u = _u_call(x, W, dinv)
    accp = msg_k(u, src, dst.reshape(NW, NCHUNK, CH))
    degA, tot = _rowsum_call(A)
    return _final_call(
        u, accp, dinv, degA, tot,
        weight.reshape(1, 1), temperature.reshape(1, 1), b.reshape(1, D),
    )


# speedup vs baseline: 1.0441x; 1.0021x over previous
import functools

import jax
import jax.numpy as jnp
from jax import lax
from jax.experimental import pallas as pl
from jax.experimental.pallas import tpu as pltpu
from jax.experimental.pallas import tpu_sc as plsc

N = 10000
E = 160000
D = 128

NC = 2
NS = 16
NW = NC * NS
EPW = E // NW
CH = 125
NCHUNK = EPW // CH


def _deg_body(dst_hbm, out_hbm, deg_v, idx_v):
    c = lax.axis_index("c")
    s = lax.axis_index("s")
    wid = s * NC + c

    zeros16 = jnp.zeros((16,), jnp.float32)

    def zero_body(i, _):
        deg_v[pl.ds(i * 16, 16)] = zeros16
        return 0

    lax.fori_loop(0, N // 16, zero_body, 0)

    pltpu.sync_copy(dst_hbm.at[wid], idx_v)

    ones16 = jnp.ones((16,), jnp.float32)
    nfull = EPW // 16

    def scat_body(i, _):
        idx = idx_v[pl.ds(i * 16, 16)]
        plsc.addupdate_scatter(deg_v, [idx], ones16)
        return 0

    lax.fori_loop(0, nfull, scat_body, 0)

    rem = EPW - nfull * 16
    if rem:
        idx = idx_v[pl.ds(EPW - 16, 16)]
        mask = lax.iota(jnp.int32, 16) >= (16 - rem)
        plsc.addupdate_scatter(deg_v, [idx], ones16, mask=mask)

    pltpu.sync_copy(deg_v, out_hbm.at[wid])


def _msg_body(u_hbm, src_hbm, dst_hbm, out_hbm, src_v, dst_v, rows0_v,
              acc_sh, sem0):
    c = lax.axis_index("c")
    s = lax.axis_index("s")
    wid = s * NC + c

    zeros16 = jnp.zeros((16,), jnp.float32)

    def zrow(r, _):
        for j in range(D // 16):
            rows0_v[r, pl.ds(j * 16, 16)] = zeros16
        return 0

    lax.fori_loop(0, CH, zrow, 0)

    @pl.when(s < 10)
    def _():
        for k in range(8):
            pltpu.sync_copy(rows0_v, acc_sh.at[pl.ds(s * 1000 + k * CH, CH)])

    pltpu.sync_copy(src_hbm.at[wid], src_v)
    pltpu.sync_copy(dst_hbm.at[wid], dst_v)

    plsc.subcore_barrier()

    def chunk_body(j, _):
        pltpu.async_copy(u_hbm.at[src_v.at[j]], rows0_v, sem0).wait()
        pltpu.sync_copy(rows0_v, acc_sh.at[dst_v.at[j]], add=True)
        return 0

    lax.fori_loop(0, NCHUNK, chunk_body, 0)

    plsc.subcore_barrier()

    @pl.when(s < 10)
    def _():
        pltpu.sync_copy(
            acc_sh.at[pl.ds(s * 1000, 1000)],
            out_hbm.at[c, pl.ds(s * 1000, 1000)],
        )


@functools.cache
def _sc_kernels():
    mesh = plsc.VectorSubcoreMesh(
        core_axis_name="c", subcore_axis_name="s", num_cores=NC, num_subcores=NS
    )
    params = pltpu.CompilerParams(needs_layout_passes=False)
    deg_k = pl.kernel(
        _deg_body,
        out_type=jax.ShapeDtypeStruct((NW, N), jnp.float32),
        mesh=mesh,
        compiler_params=params,
        scratch_types=[
            pltpu.VMEM((N,), jnp.float32),
            pltpu.VMEM((EPW,), jnp.int32),
        ],
    )
    msg_k = pl.kernel(
        _msg_body,
        out_type=jax.ShapeDtypeStruct((NC, N, D), jnp.float32),
        mesh=mesh,
        compiler_params=params,
        scratch_types=[
            pltpu.VMEM((NCHUNK, CH), jnp.int32),
            pltpu.VMEM((NCHUNK, CH), jnp.int32),
            pltpu.VMEM((CH, D), jnp.float32),
            pltpu.VMEM_SHARED((N, D), jnp.float32),
            pltpu.SemaphoreType.DMA,
        ],
    )
    return deg_k, msg_k


_RB = 1000


def _dinv_body(degp_ref, dinv_ref):
    deg = jnp.sum(degp_ref[...], axis=0) + 1.0
    dinv_ref[...] = lax.rsqrt(deg)[:, None]


def _dinv_call(degp):
    return pl.pallas_call(
        _dinv_body,
        out_shape=jax.ShapeDtypeStruct((N, 1), jnp.float32),
    )(degp)


def _u_body(x_ref, w_ref, dinv_ref, u_ref):
    h = jnp.dot(x_ref[...], w_ref[...], preferred_element_type=jnp.float32)
    u_ref[...] = h * dinv_ref[...]


def _u_call(x, W, dinv):
    return pl.pallas_call(
        _u_body,
        grid=(N // _RB,),
        in_specs=[
            pl.BlockSpec((_RB, D), lambda i: (i, 0)),
            pl.BlockSpec((D, D), lambda i: (0, 0)),
            pl.BlockSpec((_RB, 1), lambda i: (i, 0)),
        ],
        out_specs=pl.BlockSpec((_RB, D), lambda i: (i, 0)),
        out_shape=jax.ShapeDtypeStruct((N, D), jnp.float32),
    )(x, W, dinv)


_RSB = 200


def _rowsum_body(a_ref, degA_ref, tot_ref):
    i = pl.program_id(0)
    part = jnp.sum(a_ref[...], axis=1, keepdims=True)
    degA_ref[...] = part
    blk = jnp.sum(part).reshape(1, 1)

    @pl.when(i == 0)
    def _():
        tot_ref[...] = blk

    @pl.when(i > 0)
    def _():
        tot_ref[...] += blk


def _rowsum_call(A):
    return pl.pallas_call(
        _rowsum_body,
        grid=(N // _RSB,),
        in_specs=[pl.BlockSpec((_RSB, N), lambda i: (i, 0))],
        out_specs=[
            pl.BlockSpec((_RSB, 1), lambda i: (i, 0)),
            pl.BlockSpec((1, 1), lambda i: (0, 0)),
        ],
        out_shape=[
            jax.ShapeDtypeStruct((N, 1), jnp.float32),
            jax.ShapeDtypeStruct((1, 1), jnp.float32),
        ],
        compiler_params=pltpu.CompilerParams(
            dimension_semantics=("arbitrary",),
        ),
    )(A)


def _final_body(u_ref, acc_ref, dinv_ref, degA_ref, tot_ref, w_ref, t_ref,
                b_ref, out_ref):
    gcn = dinv_ref[...] * (acc_ref[0] + acc_ref[1] + u_ref[...]) + b_ref[...]

    T = t_ref[...]
    z = gcn / T
    m = jnp.max(z, axis=1, keepdims=True)
    e = jnp.exp(z - m)
    ssum = jnp.sum(e, axis=1, keepdims=True)
    p = e / ssum
    logp = jnp.log(p + 1e-12)
    H = -jnp.sum(p * logp, axis=1, keepdims=True)
    g = -(p * (logp + H)) / T
    wn = degA_ref[...] / (tot_ref[...] + 1e-12)
    out_ref[...] = gcn + w_ref[...] * (g * wn)


def _final_call(u, accp, dinv, degA, tot, weight, temperature, b):
    return pl.pallas_call(
        _final_body,
        grid=(N // _RB,),
        in_specs=[
            pl.BlockSpec((_RB, D), lambda i: (i, 0)),
            pl.BlockSpec((NC, _RB, D), lambda i: (0, i, 0)),
            pl.BlockSpec((_RB, 1), lambda i: (i, 0)),
            pl.BlockSpec((_RB, 1), lambda i: (i, 0)),
            pl.BlockSpec((1, 1), lambda i: (0, 0)),
            pl.BlockSpec((1, 1), lambda i: (0, 0)),
            pl.BlockSpec((1, 1), lambda i: (0, 0)),
            pl.BlockSpec((1, D), lambda i: (0, 0)),
        ],
        out_specs=pl.BlockSpec((_RB, D), lambda i: (i, 0)),
        out_shape=jax.ShapeDtypeStruct((N, D), jnp.float32),
    )(u, accp, dinv, degA, tot, weight, temperature, b)


@jax.jit
def kernel(x, edge_index, A, weight, temperature, W, b):
    src = edge_index[0].reshape(NW, NCHUNK, CH)
    dst = edge_index[1]

    deg_k, msg_k = _sc_kernels()
    degp = deg_k(dst.reshape(NW, EPW))
    dinv = _dinv_call(degp)
    u = _u_call(x, W, dinv)
    accp = msg_k(u, src, dst.reshape(NW, NCHUNK, CH))
    degA, tot = _rowsum_call(A)
    return _final_call(
        u, accp, dinv, degA, tot,
        weight.reshape(1, 1), temperature.reshape(1, 1), b.reshape(1, D),
    )
